# Initial kernel scaffold; baseline (speedup 1.0000x reference)
import functools

import jax
import jax.numpy as jnp
from jax import lax
from jax.experimental import pallas as pl
from jax.experimental.pallas import tpu as pltpu
from jax.experimental.pallas import tpu_sc as plsc

NUM_CORES = 2
NUM_SUBCORES = 16
NUM_WORKERS = NUM_CORES * NUM_SUBCORES

BATCH = 16384
HIST_LEN = 50
DIM = 64
B_TOTAL = BATCH * HIST_LEN
B_PER_W = B_TOTAL // NUM_WORKERS
CHUNK = 512
N_CHUNKS = B_PER_W // CHUNK

_MESH = plsc.VectorSubcoreMesh(
    core_axis_name="c",
    subcore_axis_name="s",
    num_cores=NUM_CORES,
    num_subcores=NUM_SUBCORES,
)


@functools.partial(
    pl.kernel,
    out_type=jax.ShapeDtypeStruct((B_TOTAL, DIM), jnp.float32),
    mesh=_MESH,
    scratch_types=[
        pltpu.VMEM((CHUNK,), jnp.int32),
        pltpu.VMEM((CHUNK, DIM), jnp.float32),
        pltpu.SemaphoreType.DMA,
    ],
)
def _gather_kernel(table_hbm, idx_hbm, out_hbm, idx_v, rows_v, sem):
    wid = lax.axis_index("s") * NUM_CORES + lax.axis_index("c")
    base = wid * B_PER_W

    def body(j, _):
        off = base + j * CHUNK
        pltpu.sync_copy(idx_hbm.at[pl.ds(off, CHUNK)], idx_v)
        pltpu.async_copy(table_hbm.at[idx_v], rows_v, sem).wait()
        pltpu.sync_copy(rows_v, out_hbm.at[pl.ds(off, CHUNK)])
        return 0

    lax.fori_loop(0, N_CHUNKS, body, 0)


def kernel(x, embedding):
    idx = x.reshape(B_TOTAL)
    rows = _gather_kernel(embedding, idx)
    return rows.reshape(BATCH, HIST_LEN, DIM)


# speedup vs baseline: 1.7973x; 1.7973x over previous
import functools

import jax
import jax.numpy as jnp
from jax import lax
from jax.experimental import pallas as pl
from jax.experimental.pallas import tpu as pltpu
from jax.experimental.pallas import tpu_sc as plsc

NUM_CORES = 2
NUM_SUBCORES = 16
NUM_WORKERS = NUM_CORES * NUM_SUBCORES

BATCH = 16384
HIST_LEN = 50
DIM = 64
B_TOTAL = BATCH * HIST_LEN
B_PER_W = B_TOTAL // NUM_WORKERS
CHUNK = 512
N_CHUNKS = B_PER_W // CHUNK

_MESH = plsc.VectorSubcoreMesh(
    core_axis_name="c",
    subcore_axis_name="s",
    num_cores=NUM_CORES,
    num_subcores=NUM_SUBCORES,
)


@functools.partial(
    pl.kernel,
    out_type=jax.ShapeDtypeStruct((B_TOTAL, DIM), jnp.float32),
    mesh=_MESH,
    scratch_types=[
        pltpu.VMEM((CHUNK,), jnp.int32),
        pltpu.VMEM((CHUNK, DIM), jnp.float32),
        pltpu.SemaphoreType.DMA,
    ],
    compiler_params=pltpu.CompilerParams(use_tc_tiling_on_sc=False),
)
def _gather_kernel(table_hbm, idx_hbm, out_hbm, idx_v, rows_v, sem):
    wid = lax.axis_index("s") * NUM_CORES + lax.axis_index("c")
    base = wid * B_PER_W

    def body(j, _):
        off = base + j * CHUNK
        pltpu.sync_copy(idx_hbm.at[pl.ds(off, CHUNK)], idx_v)
        pltpu.async_copy(table_hbm.at[idx_v], rows_v, sem).wait()
        pltpu.sync_copy(rows_v, out_hbm.at[pl.ds(off, CHUNK)])
        return 0

    lax.fori_loop(0, N_CHUNKS, body, 0)


def kernel(x, embedding):
    idx = x.reshape(B_TOTAL)
    rows = _gather_kernel(embedding, idx)
    return rows.reshape(BATCH, HIST_LEN, DIM)


# speedup vs baseline: 1.8752x; 1.0433x over previous
import functools

import jax
import jax.numpy as jnp
from jax import lax
from jax.experimental import pallas as pl
from jax.experimental.pallas import tpu as pltpu
from jax.experimental.pallas import tpu_sc as plsc

NUM_CORES = 2
NUM_SUBCORES = 16
NUM_WORKERS = NUM_CORES * NUM_SUBCORES

BATCH = 16384
HIST_LEN = 50
DIM = 64
B_TOTAL = BATCH * HIST_LEN
B_PER_W = B_TOTAL // NUM_WORKERS
CHUNK = 512
N_CHUNKS = B_PER_W // CHUNK
NBUF = 2
N_GROUPS = N_CHUNKS // NBUF

_MESH = plsc.VectorSubcoreMesh(
    core_axis_name="c",
    subcore_axis_name="s",
    num_cores=NUM_CORES,
    num_subcores=NUM_SUBCORES,
)


@functools.partial(
    pl.kernel,
    out_type=jax.ShapeDtypeStruct((B_TOTAL, DIM), jnp.float32),
    mesh=_MESH,
    scratch_types=[
        pltpu.VMEM((B_PER_W,), jnp.int32),
        pltpu.VMEM((CHUNK, DIM), jnp.float32),
        pltpu.VMEM((CHUNK, DIM), jnp.float32),
        pltpu.SemaphoreType.DMA,
        pltpu.SemaphoreType.DMA,
    ],
    compiler_params=pltpu.CompilerParams(use_tc_tiling_on_sc=False),
)
def _gather_kernel(table_hbm, idx_hbm, out_hbm, idx_all, rows0, rows1,
                   sem0, sem1):
    wid = lax.axis_index("s") * NUM_CORES + lax.axis_index("c")
    base = wid * B_PER_W
    rows = [rows0, rows1]
    sems = [sem0, sem1]

    pltpu.sync_copy(idx_hbm.at[pl.ds(base, B_PER_W)], idx_all)

    def start_gather(j, b):
        pltpu.async_copy(
            table_hbm.at[idx_all.at[pl.ds(j * CHUNK, CHUNK)]], rows[b],
            sems[b])

    def wait_gather(j, b):
        pltpu.make_async_copy(
            table_hbm.at[idx_all.at[pl.ds(j * CHUNK, CHUNK)]], rows[b],
            sems[b]).wait()

    start_gather(0, 0)

    def group(g, _):
        for b in range(NBUF):
            j = g * NBUF + b
            bn = (b + 1) % NBUF

            @pl.when(j + 1 < N_CHUNKS)
            def _():
                start_gather(j + 1, bn)

            wait_gather(j, b)
            pltpu.sync_copy(rows[b], out_hbm.at[pl.ds(base + j * CHUNK,
                                                      CHUNK)])
        return 0

    lax.fori_loop(0, N_GROUPS, group, 0)


def kernel(x, embedding):
    idx = x.reshape(B_TOTAL)
    rows = _gather_kernel(embedding, idx)
    return rows.reshape(BATCH, HIST_LEN, DIM)


# speedup vs baseline: 1.8753x; 1.0001x over previous
import functools

import jax
import jax.numpy as jnp
from jax import lax
from jax.experimental import pallas as pl
from jax.experimental.pallas import tpu as pltpu
from jax.experimental.pallas import tpu_sc as plsc

NUM_CORES = 2
NUM_SUBCORES = 16
NUM_WORKERS = NUM_CORES * NUM_SUBCORES

BATCH = 16384
HIST_LEN = 50
DIM = 64
B_TOTAL = BATCH * HIST_LEN
B_PER_W = B_TOTAL // NUM_WORKERS
CHUNK = 256
N_CHUNKS = B_PER_W // CHUNK
NBUF = 4
N_GROUPS = N_CHUNKS // NBUF

_MESH = plsc.VectorSubcoreMesh(
    core_axis_name="c",
    subcore_axis_name="s",
    num_cores=NUM_CORES,
    num_subcores=NUM_SUBCORES,
)


@functools.partial(
    pl.kernel,
    out_type=jax.ShapeDtypeStruct((B_TOTAL, DIM), jnp.float32),
    mesh=_MESH,
    scratch_types=(
        [pltpu.VMEM((B_PER_W,), jnp.int32)]
        + [pltpu.VMEM((CHUNK, DIM), jnp.float32) for _ in range(NBUF)]
        + [pltpu.SemaphoreType.DMA for _ in range(2 * NBUF)]
    ),
    compiler_params=pltpu.CompilerParams(use_tc_tiling_on_sc=False),
)
def _gather_kernel(table_hbm, idx_hbm, out_hbm, idx_all, *bufs):
    rows = list(bufs[:NBUF])
    sem_g = list(bufs[NBUF:2 * NBUF])
    sem_w = list(bufs[2 * NBUF:])

    wid = lax.axis_index("s") * NUM_CORES + lax.axis_index("c")
    base = wid * B_PER_W

    pltpu.sync_copy(idx_hbm.at[pl.ds(base, B_PER_W)], idx_all)

    def gather_desc(j, b):
        return pltpu.make_async_copy(
            table_hbm.at[idx_all.at[pl.ds(j * CHUNK, CHUNK)]], rows[b],
            sem_g[b])

    def write_desc(j, b):
        return pltpu.make_async_copy(
            rows[b], out_hbm.at[pl.ds(base + j * CHUNK, CHUNK)], sem_w[b])

    for b in range(NBUF - 1):
        gather_desc(b, b).start()

    def group(g, _):
        for b in range(NBUF):
            j = g * NBUF + b
            bn = (b + NBUF - 1) % NBUF
            jn = j + NBUF - 1

            @pl.when(jn < N_CHUNKS)
            def _():
                @pl.when(jn >= NBUF)
                def _():
                    write_desc(jn - NBUF, bn).wait()

                gather_desc(jn, bn).start()

            gather_desc(j, b).wait()
            write_desc(j, b).start()
        return 0

    lax.fori_loop(0, N_GROUPS, group, 0)

    for b in range(NBUF):
        write_desc(N_CHUNKS - NBUF + b, b).wait()


def kernel(x, embedding):
    idx = x.reshape(B_TOTAL)
    rows = _gather_kernel(embedding, idx)
    return rows.reshape(BATCH, HIST_LEN, DIM)
